# baseline (device time: 448114 ns/iter reference)
import jax

jax.config.update("jax_compilation_cache_dir", "/tmp/jax_cache")
jax.config.update("jax_persistent_cache_min_entry_size_bytes", -1)
jax.config.update("jax_persistent_cache_min_compile_time_secs", 0)

import jax.numpy as jnp
from jax import lax
from jax.experimental import pallas as pl
from jax.experimental.pallas import tpu as pltpu

B = 4
S = 1024
S_HALF = 512
R = 256
K = 2048
N = 4096
N_CHUNK = B * (S_HALF // R)
N_SEND = 3


def kernel(O, Wo):
    O2 = O.reshape(B, S, K)

    def body(o_ref, wo_ref, out_ref, o_buf, s_buf, a_buf,
             o_sems, ld_sems, store_sems, send_sems, recv_sems):
        my_x = lax.axis_index("x")
        my_y = lax.axis_index("y")
        peer = (my_x, 1 - my_y)
        my_lo = my_y * S_HALF
        peer_lo = (1 - my_y) * S_HALF

        def o_load_desc(half_lo, c):
            b, j = c // 2, c % 2
            return pltpu.make_async_copy(
                o_ref.at[b, pl.ds(half_lo + j * R, R), :],
                o_buf.at[c % 2],
                o_sems.at[c % 2],
            )

        def remote_desc(c):
            b, j = c // 2, c % 2
            return pltpu.make_async_remote_copy(
                src_ref=s_buf.at[c % N_SEND],
                dst_ref=out_ref.at[b, pl.ds(j * R, R), :],
                send_sem=send_sems.at[c % N_SEND],
                recv_sem=recv_sems.at[c],
                device_id=peer,
                device_id_type=pl.DeviceIdType.MESH,
            )

        def ld_desc(c):
            b, j = c // 2, c % 2
            return pltpu.make_async_copy(
                out_ref.at[b, pl.ds(j * R, R), :],
                a_buf.at[c % 2],
                ld_sems.at[c % 2],
            )

        def store_desc(c):
            b, j = c // 2, c % 2
            return pltpu.make_async_copy(
                a_buf.at[c % 2],
                out_ref.at[b, pl.ds(j * R, R), :],
                store_sems.at[c % 2],
            )

        o_load_desc(peer_lo, 0).start()

        barrier = pltpu.get_barrier_semaphore()
        pl.semaphore_signal(
            barrier, inc=1, device_id=peer, device_id_type=pl.DeviceIdType.MESH
        )
        pl.semaphore_wait(barrier, 1)

        def phase1(c, carry):
            o_load_desc(peer_lo, c).wait()

            @pl.when(c < N_CHUNK - 1)
            def _():
                o_load_desc(peer_lo, c + 1).start()

            @pl.when(c == N_CHUNK - 1)
            def _():
                o_load_desc(my_lo, 0).start()

            @pl.when(c >= N_SEND)
            def _():
                remote_desc(c - N_SEND).wait_send()

            s_buf[c % N_SEND] = jnp.dot(
                o_buf[c % 2], wo_ref[...], preferred_element_type=jnp.float32
            )
            remote_desc(c).start()
            return carry

        lax.fori_loop(0, N_CHUNK, phase1, 0)

        def phase2(c, carry):
            o_load_desc(my_lo, c).wait()

            @pl.when(c < N_CHUNK - 1)
            def _():
                o_load_desc(my_lo, c + 1).start()

            remote_desc(c).wait_recv()

            @pl.when(c >= 2)
            def _():
                store_desc(c - 2).wait()

            ld_desc(c).start()
            part = jnp.dot(
                o_buf[c % 2], wo_ref[...], preferred_element_type=jnp.float32
            )
            ld_desc(c).wait()
            a_buf[c % 2] = a_buf[c % 2] + part
            store_desc(c).start()
            return carry

        lax.fori_loop(0, N_CHUNK, phase2, 0)

        for c in range(N_CHUNK - N_SEND, N_CHUNK):
            remote_desc(c).wait_send()
        for c in range(N_CHUNK - 2, N_CHUNK):
            store_desc(c).wait()

    return pl.pallas_call(
        body,
        out_shape=jax.ShapeDtypeStruct((B, S_HALF, N), jnp.float32),
        in_specs=[
            pl.BlockSpec(memory_space=pl.ANY),
            pl.BlockSpec(memory_space=pltpu.VMEM),
        ],
        out_specs=pl.BlockSpec(memory_space=pl.ANY),
        scratch_shapes=[
            pltpu.VMEM((2, R, K), jnp.float32),
            pltpu.VMEM((N_SEND, R, N), jnp.float32),
            pltpu.VMEM((2, R, N), jnp.float32),
            pltpu.SemaphoreType.DMA((2,)),
            pltpu.SemaphoreType.DMA((2,)),
            pltpu.SemaphoreType.DMA((2,)),
            pltpu.SemaphoreType.DMA((N_SEND,)),
            pltpu.SemaphoreType.DMA((N_CHUNK,)),
        ],
        compiler_params=pltpu.CompilerParams(
            collective_id=0, vmem_limit_bytes=63 * 1024 * 1024
        ),
    )(O2, Wo)


# device time: 442987 ns/iter; 1.0116x vs baseline; 1.0116x over previous
import jax

jax.config.update("jax_compilation_cache_dir", "/tmp/jax_cache")
jax.config.update("jax_persistent_cache_min_entry_size_bytes", -1)
jax.config.update("jax_persistent_cache_min_compile_time_secs", 0)

import jax.numpy as jnp
from jax import lax
from jax.experimental import pallas as pl
from jax.experimental.pallas import tpu as pltpu

B = 4
S = 1024
S_HALF = 512
R = 256
K = 2048
N = 4096
N_CHUNK = B * (S_HALF // R)
N_SEND = 3


def kernel(O, Wo):
    O2 = O.reshape(B, S, K)

    def body(o_ref, wo_ref, out_ref, o_buf, s_buf, a_buf,
             o_sems, ld_sems, store_sems, send_sems, recv_sems):
        my_x = lax.axis_index("x")
        my_y = lax.axis_index("y")
        peer = (my_x, 1 - my_y)
        my_lo = my_y * S_HALF
        peer_lo = (1 - my_y) * S_HALF

        def o_load_desc(half_lo, c):
            b, j = c // 2, c % 2
            return pltpu.make_async_copy(
                o_ref.at[b, pl.ds(half_lo + j * R, R), :],
                o_buf.at[c % 2],
                o_sems.at[c % 2],
            )

        def remote_desc(c):
            b, j = c // 2, c % 2
            return pltpu.make_async_remote_copy(
                src_ref=s_buf.at[c % N_SEND],
                dst_ref=out_ref.at[b, pl.ds(j * R, R), :],
                send_sem=send_sems.at[c % N_SEND],
                recv_sem=recv_sems.at[c],
                device_id=peer,
                device_id_type=pl.DeviceIdType.MESH,
            )

        def ld_desc(c):
            b, j = c // 2, c % 2
            return pltpu.make_async_copy(
                out_ref.at[b, pl.ds(j * R, R), :],
                a_buf.at[c % 2],
                ld_sems.at[c % 2],
            )

        def store_desc(c):
            b, j = c // 2, c % 2
            return pltpu.make_async_copy(
                a_buf.at[c % 2],
                out_ref.at[b, pl.ds(j * R, R), :],
                store_sems.at[c % 2],
            )

        o_load_desc(peer_lo, 0).start()

        barrier = pltpu.get_barrier_semaphore()
        pl.semaphore_signal(
            barrier, inc=1, device_id=peer, device_id_type=pl.DeviceIdType.MESH
        )
        pl.semaphore_wait(barrier, 1)

        def phase1(c, carry):
            o_load_desc(peer_lo, c).wait()

            @pl.when(c < N_CHUNK - 1)
            def _():
                o_load_desc(peer_lo, c + 1).start()

            @pl.when(c == N_CHUNK - 1)
            def _():
                o_load_desc(my_lo, 0).start()

            @pl.when(c >= N_SEND)
            def _():
                remote_desc(c - N_SEND).wait_send()

            s_buf[c % N_SEND] = jnp.dot(
                o_buf[c % 2], wo_ref[...], preferred_element_type=jnp.float32
            )
            remote_desc(c).start()
            return carry

        lax.fori_loop(0, N_CHUNK, phase1, 0)

        def phase2(c, carry):
            o_load_desc(my_lo, c).wait()

            @pl.when(c < N_CHUNK - 1)
            def _():
                o_load_desc(my_lo, c + 1).start()

            @pl.when(c >= 2)
            def _():
                store_desc(c - 2).wait()

            part = jnp.dot(
                o_buf[c % 2], wo_ref[...], preferred_element_type=jnp.float32
            )
            remote_desc(c).wait_recv()
            ld_desc(c).start()
            ld_desc(c).wait()
            a_buf[c % 2] = a_buf[c % 2] + part
            store_desc(c).start()
            return carry

        lax.fori_loop(0, N_CHUNK, phase2, 0)

        for c in range(N_CHUNK - N_SEND, N_CHUNK):
            remote_desc(c).wait_send()
        for c in range(N_CHUNK - 2, N_CHUNK):
            store_desc(c).wait()

    return pl.pallas_call(
        body,
        out_shape=jax.ShapeDtypeStruct((B, S_HALF, N), jnp.float32),
        in_specs=[
            pl.BlockSpec(memory_space=pl.ANY),
            pl.BlockSpec(memory_space=pltpu.VMEM),
        ],
        out_specs=pl.BlockSpec(memory_space=pl.ANY),
        scratch_shapes=[
            pltpu.VMEM((2, R, K), jnp.float32),
            pltpu.VMEM((N_SEND, R, N), jnp.float32),
            pltpu.VMEM((2, R, N), jnp.float32),
            pltpu.SemaphoreType.DMA((2,)),
            pltpu.SemaphoreType.DMA((2,)),
            pltpu.SemaphoreType.DMA((2,)),
            pltpu.SemaphoreType.DMA((N_SEND,)),
            pltpu.SemaphoreType.DMA((N_CHUNK,)),
        ],
        compiler_params=pltpu.CompilerParams(
            collective_id=0, vmem_limit_bytes=63 * 1024 * 1024
        ),
    )(O2, Wo)
